# baseline (speedup 1.0000x reference)
import functools

import jax
import jax.numpy as jnp
from jax import lax
from jax.experimental import pallas as pl
from jax.experimental.pallas import tpu as pltpu
from jax.experimental.pallas import tpu_sc as plsc

NUM_USERS = 128
NUM_ITEMS = 100000
NUM_GENDER = 3
NUM_AGE = 7
NUM_OCC = 21
EMB = 64
HID = 128
FEAT = 16
B = 4096
H = 50
H_PAD = 56
H_STAGE = 128

NUM_CORES = 2
NUM_SUBCORES = 16
NW = NUM_CORES * NUM_SUBCORES
ROWS_PER_W = B // NW


NBUF = 6
NCHUNK = ROWS_PER_W


def _sc_body(idx_hbm, mask_hbm, pos_hbm, neg_hbm, item_hbm2,
             hist_out, pos_out, neg_out,
             idx_v, mask_v, buf0, buf1, buf2, buf3, buf4, buf5,
             acc_v, pidx_v, nidx_v, prows_v, nrows_v,
             gsem0, gsem1, gsem2, gsem3, gsem4, gsem5, psem, nsem):
    bufs = (buf0, buf1, buf2, buf3, buf4, buf5)
    gsems = (gsem0, gsem1, gsem2, gsem3, gsem4, gsem5)
    item_hbm = item_hbm2
    cid = lax.axis_index("c")
    sid = lax.axis_index("s")
    wid = sid * NUM_CORES + cid
    base = wid * ROWS_PER_W

    pltpu.sync_copy(idx_hbm.at[pl.ds(base, ROWS_PER_W)], idx_v)

    def gstart(c, buf, sem):
        src = item_hbm.at[idx_v.at[c, pl.ds(0, H_PAD)]]
        pltpu.make_async_copy(src, buf, sem).start()

    def gwait(buf, sem):
        pltpu.make_async_copy(
            item_hbm.at[idx_v.at[0, pl.ds(0, H_PAD)]], buf, sem).wait()

    for b in range(NBUF):
        gstart(b, bufs[b], gsems[b])
    pltpu.sync_copy(pos_hbm.at[pl.ds(base, ROWS_PER_W)], pidx_v)
    pltpu.make_async_copy(item_hbm.at[pidx_v], prows_v, psem).start()
    pltpu.sync_copy(neg_hbm.at[pl.ds(base, ROWS_PER_W)], nidx_v)
    pltpu.make_async_copy(item_hbm.at[nidx_v], nrows_v, nsem).start()
    pltpu.sync_copy(mask_hbm.at[pl.ds(base, ROWS_PER_W)], mask_v)

    def compute(r, rows):
        accs = [jnp.zeros((16,), jnp.float32) for _ in range(4)]
        msum = jnp.zeros((16,), jnp.float32)
        lane = lax.iota(jnp.int32, 16)
        for hb, lo in ((0, 0), (16, 0), (32, 0), (34, 14)):
            mvec = mask_v[r, pl.ds(hb, 16)]
            msum = msum + (mvec if lo == 0 else
                           jnp.where(lane >= lo, mvec, 0.0))
            for h2 in range(lo, 16):
                h = hb + h2
                wv = jnp.full((16,), mvec[h2], jnp.float32)
                for j in range(4):
                    accs[j] = accs[j] + rows[h, pl.ds(16 * j, 16)] * wv
        tvec = jnp.full((16,), jnp.sum(msum, axis=0), jnp.float32)
        wv = 1.0 / jnp.maximum(tvec, 1.0)
        for j, a in enumerate(accs):
            acc_v[r, pl.ds(16 * j, 16)] = a * wv

    def loop_body(i, _):
        for b in range(NBUF):
            c = NBUF * i + b
            gwait(bufs[b], gsems[b])
            compute(c, bufs[b])

            @pl.when(c + NBUF < NCHUNK)
            def _s():
                gstart(c + NBUF, bufs[b], gsems[b])
        return ()

    lax.fori_loop(0, NCHUNK // NBUF, loop_body, ())
    for b in range(NCHUNK % NBUF):
        c = (NCHUNK // NBUF) * NBUF + b
        gwait(bufs[b], gsems[b])
        compute(c, bufs[b])

    pltpu.sync_copy(acc_v, hist_out.at[pl.ds(base, ROWS_PER_W)])
    pltpu.make_async_copy(item_hbm.at[pidx_v], prows_v, psem).wait()
    pltpu.sync_copy(prows_v, pos_out.at[pl.ds(base, ROWS_PER_W)])
    pltpu.make_async_copy(item_hbm.at[nidx_v], nrows_v, nsem).wait()
    pltpu.sync_copy(nrows_v, neg_out.at[pl.ds(base, ROWS_PER_W)])


def _sc_pool(idx_pad, mask_pad, pos_ids, neg_ids, item_flat):
    f32 = jnp.float32
    mesh = plsc.VectorSubcoreMesh(core_axis_name="c", subcore_axis_name="s")
    kern = functools.partial(
        pl.kernel, mesh=mesh,
        compiler_params=pltpu.CompilerParams(use_tc_tiling_on_sc=False, needs_layout_passes=False),
        out_type=[jax.ShapeDtypeStruct((B, EMB), f32) for _ in range(3)],
        scratch_types=(
            [pltpu.VMEM((ROWS_PER_W, H_STAGE), jnp.int32)]
            + [pltpu.VMEM((ROWS_PER_W, H_STAGE), f32)]
            + [pltpu.VMEM((H_PAD, EMB), f32)
               for _ in range(NBUF)]
            + [pltpu.VMEM((ROWS_PER_W, EMB), f32)]
            + [pltpu.VMEM((ROWS_PER_W,), jnp.int32)] * 2
            + [pltpu.VMEM((ROWS_PER_W, EMB), f32)] * 2
            + [pltpu.SemaphoreType.DMA] * (NBUF + 2)
        ),
    )(_sc_body)
    return kern(idx_pad, mask_pad, pos_ids, neg_ids, item_flat)


TCB = 1024


def _tc_body(ue_ref, uo_ref, hmean_ref, pos_ref, neg_ref,
             uemb_ref, gemb_ref, aemb_ref, oemb_ref,
             gid_ref, aid_ref, oid_ref,
             W1_ref, b1_ref, W2_ref, b2_ref,
             pe_out, po_out, ne_out, no_out):
    f32 = jnp.float32
    B2 = TCB // 2

    def dot(a, b):
        return jnp.dot(a, b, preferred_element_type=f32)

    def onehot(ids, n):
        return (ids[:, None] == lax.broadcasted_iota(
            jnp.int32, (ids.shape[0], n), 1)).astype(f32)

    g = dot(onehot(gid_ref[:], NUM_GENDER), gemb_ref[:])
    a = dot(onehot(aid_ref[:], NUM_AGE), aemb_ref[:])
    o = dot(onehot(oid_ref[:], NUM_OCC), oemb_ref[:])
    T = (dot(uemb_ref[:], W1_ref[0:EMB, :])
         + dot(g, W1_ref[EMB * 2:EMB * 2 + FEAT, :])
         + dot(a, W1_ref[EMB * 2 + FEAT:EMB * 2 + 2 * FEAT, :])
         + dot(o, W1_ref[EMB * 2 + 2 * FEAT:EMB * 2 + 3 * FEAT, :])
         + b1_ref[:][None, :])

    hm = hmean_ref[:].reshape(B2, 2 * EMB)
    pp = pos_ref[:].reshape(B2, 2 * EMB)
    nn = neg_ref[:].reshape(B2, 2 * EMB)
    W1h = W1_ref[EMB:EMB * 2, :]

    def tower(u1h, hmean):
        hidden = jnp.maximum(dot(u1h, T) + dot(hmean, W1h), 0.0)
        return dot(hidden, W2_ref[:]) + b2_ref[:][None, :]

    uv_e = tower(onehot(ue_ref[:], NUM_USERS), hm[:, 0:EMB])
    uv_o = tower(onehot(uo_ref[:], NUM_USERS), hm[:, EMB:2 * EMB])
    pe_out[:] = jnp.sum(uv_e * pp[:, 0:EMB], axis=1)
    po_out[:] = jnp.sum(uv_o * pp[:, EMB:2 * EMB], axis=1)
    ne_out[:] = jnp.sum(uv_e * nn[:, 0:EMB], axis=1)
    no_out[:] = jnp.sum(uv_o * nn[:, EMB:2 * EMB], axis=1)


def kernel(user_ids, history_item_ids, history_mask, pos_item_ids,
           neg_item_ids, user_emb, item_emb, gender_emb, age_emb, occ_emb,
           W1, b1, W2, b2, gender_ids, age_ids, occupation_ids):
    f32 = jnp.float32
    pad = H_STAGE - H
    pad_idx = (jnp.arange(B * pad, dtype=jnp.int32) % jnp.int32(99991)
               ).reshape(B, pad)
    idx_pad = jnp.concatenate([history_item_ids, pad_idx], axis=1)
    mask_pad = jnp.pad(history_mask, ((0, 0), (0, pad)))

    hist_sum, pos_rows, neg_rows = _sc_pool(
        idx_pad, mask_pad, pos_item_ids, neg_item_ids,
        item_emb)

    full = lambda a: pl.BlockSpec(a.shape, lambda i: tuple(
        0 for _ in a.shape))
    pe, po, ne, no = pl.pallas_call(
        _tc_body,
        grid=(B // TCB,),
        in_specs=[
            pl.BlockSpec((TCB // 2,), lambda i: (i,)),
            pl.BlockSpec((TCB // 2,), lambda i: (i,)),
            pl.BlockSpec((TCB * EMB,), lambda i: (i,)),
            pl.BlockSpec((TCB * EMB,), lambda i: (i,)),
            pl.BlockSpec((TCB * EMB,), lambda i: (i,)),
            full(user_emb), full(gender_emb), full(age_emb), full(occ_emb),
            full(gender_ids), full(age_ids), full(occupation_ids),
            full(W1), full(b1), full(W2), full(b2),
        ],
        out_specs=[pl.BlockSpec((TCB // 2,), lambda i: (i,))] * 4,
        out_shape=[jax.ShapeDtypeStruct((B // 2,), f32) for _ in range(4)],
    )(user_ids[0::2], user_ids[1::2], hist_sum.reshape(B * EMB),
      pos_rows.reshape(B * EMB), neg_rows.reshape(B * EMB),
      user_emb, gender_emb, age_emb, occ_emb,
      gender_ids, age_ids, occupation_ids, W1, b1, W2, b2)
    pos_s = jnp.stack([pe, po], axis=1).reshape(B)
    neg_s = jnp.stack([ne, no], axis=1).reshape(B)
    return pos_s, neg_s


# speedup vs baseline: 1.0137x; 1.0137x over previous
import functools

import jax
import jax.numpy as jnp
from jax import lax
from jax.experimental import pallas as pl
from jax.experimental.pallas import tpu as pltpu
from jax.experimental.pallas import tpu_sc as plsc

NUM_USERS = 128
NUM_ITEMS = 100000
NUM_GENDER = 3
NUM_AGE = 7
NUM_OCC = 21
EMB = 64
HID = 128
FEAT = 16
B = 4096
H = 50
H_PAD = 56
H_STAGE = 128

NUM_CORES = 2
NUM_SUBCORES = 16
NW = NUM_CORES * NUM_SUBCORES
ROWS_PER_W = B // NW


NBUF = 6
NCHUNK = ROWS_PER_W


def _sc_body(idx_hbm, mask_hbm, pos_hbm, neg_hbm, item_hbm2,
             hist_out, pos_out, neg_out,
             idx_v, mask_v, buf0, buf1, buf2, buf3, buf4, buf5,
             acc_v, pidx_v, nidx_v, prows_v, nrows_v,
             gsem0, gsem1, gsem2, gsem3, gsem4, gsem5, psem, nsem):
    bufs = (buf0, buf1, buf2, buf3, buf4, buf5)
    gsems = (gsem0, gsem1, gsem2, gsem3, gsem4, gsem5)
    item_hbm = item_hbm2
    cid = lax.axis_index("c")
    sid = lax.axis_index("s")
    wid = sid * NUM_CORES + cid
    base = wid * ROWS_PER_W

    pltpu.sync_copy(idx_hbm.at[pl.ds(base, ROWS_PER_W)], idx_v)

    def gstart(c, buf, sem):
        src = item_hbm.at[idx_v.at[c, pl.ds(0, H_PAD)]]
        pltpu.make_async_copy(src, buf, sem).start()

    def gwait(buf, sem):
        pltpu.make_async_copy(
            item_hbm.at[idx_v.at[0, pl.ds(0, H_PAD)]], buf, sem).wait()

    for b in range(NBUF):
        gstart(b, bufs[b], gsems[b])
    pltpu.sync_copy(pos_hbm.at[pl.ds(base, ROWS_PER_W)], pidx_v)
    pltpu.make_async_copy(item_hbm.at[pidx_v], prows_v, psem).start()
    pltpu.sync_copy(neg_hbm.at[pl.ds(base, ROWS_PER_W)], nidx_v)
    pltpu.make_async_copy(item_hbm.at[nidx_v], nrows_v, nsem).start()
    pltpu.sync_copy(mask_hbm.at[pl.ds(base, ROWS_PER_W)], mask_v)

    def compute(r, rows):
        accs = [jnp.zeros((16,), jnp.float32) for _ in range(4)]
        msum = jnp.zeros((16,), jnp.float32)
        lane = lax.iota(jnp.int32, 16)
        for hb, lo in ((0, 0), (16, 0), (32, 0), (34, 14)):
            mvec = mask_v[r, pl.ds(hb, 16)]
            msum = msum + (mvec if lo == 0 else
                           jnp.where(lane >= lo, mvec, 0.0))
            for h2 in range(lo, 16):
                h = hb + h2
                wv = jnp.full((16,), mvec[h2], jnp.float32)
                for j in range(4):
                    accs[j] = accs[j] + rows[h, pl.ds(16 * j, 16)] * wv
        tvec = jnp.full((16,), jnp.sum(msum, axis=0), jnp.float32)
        wv = 1.0 / jnp.maximum(tvec, 1.0)
        for j, a in enumerate(accs):
            acc_v[r, pl.ds(16 * j, 16)] = a * wv

    def loop_body(i, _):
        for b in range(NBUF):
            c = NBUF * i + b
            gwait(bufs[b], gsems[b])
            compute(c, bufs[b])

            @pl.when(c + NBUF < NCHUNK)
            def _s():
                gstart(c + NBUF, bufs[b], gsems[b])
        return ()

    lax.fori_loop(0, NCHUNK // NBUF, loop_body, ())
    for b in range(NCHUNK % NBUF):
        c = (NCHUNK // NBUF) * NBUF + b
        gwait(bufs[b], gsems[b])
        compute(c, bufs[b])

    pltpu.sync_copy(acc_v, hist_out.at[pl.ds(base, ROWS_PER_W)])
    pltpu.make_async_copy(item_hbm.at[pidx_v], prows_v, psem).wait()
    pltpu.sync_copy(prows_v, pos_out.at[pl.ds(base, ROWS_PER_W)])
    pltpu.make_async_copy(item_hbm.at[nidx_v], nrows_v, nsem).wait()
    pltpu.sync_copy(nrows_v, neg_out.at[pl.ds(base, ROWS_PER_W)])


def _sc_pool(idx_pad, mask_pad, pos_ids, neg_ids, item_flat):
    f32 = jnp.float32
    mesh = plsc.VectorSubcoreMesh(core_axis_name="c", subcore_axis_name="s")
    kern = functools.partial(
        pl.kernel, mesh=mesh,
        compiler_params=pltpu.CompilerParams(use_tc_tiling_on_sc=False, needs_layout_passes=False),
        out_type=[jax.ShapeDtypeStruct((B, EMB), f32) for _ in range(3)],
        scratch_types=(
            [pltpu.VMEM((ROWS_PER_W, H_STAGE), jnp.int32)]
            + [pltpu.VMEM((ROWS_PER_W, H_STAGE), f32)]
            + [pltpu.VMEM((H_PAD, EMB), f32)
               for _ in range(NBUF)]
            + [pltpu.VMEM((ROWS_PER_W, EMB), f32)]
            + [pltpu.VMEM((ROWS_PER_W,), jnp.int32)] * 2
            + [pltpu.VMEM((ROWS_PER_W, EMB), f32)] * 2
            + [pltpu.SemaphoreType.DMA] * (NBUF + 2)
        ),
    )(_sc_body)
    return kern(idx_pad, mask_pad, pos_ids, neg_ids, item_flat)


def _tc_body(ue_ref, uo_ref, hmean_ref, pos_ref, neg_ref,
             uemb_ref, gemb_ref, aemb_ref, oemb_ref,
             gid_ref, aid_ref, oid_ref,
             W1_ref, b1_ref, W2_ref, b2_ref,
             pe_out, po_out, ne_out, no_out):
    f32 = jnp.float32
    B2 = B // 2

    def dot(a, b):
        return jnp.dot(a, b, preferred_element_type=f32)

    def onehot(ids, n):
        return (ids[:, None] == lax.broadcasted_iota(
            jnp.int32, (ids.shape[0], n), 1)).astype(f32)

    g = dot(onehot(gid_ref[:], NUM_GENDER), gemb_ref[:])
    a = dot(onehot(aid_ref[:], NUM_AGE), aemb_ref[:])
    o = dot(onehot(oid_ref[:], NUM_OCC), oemb_ref[:])
    T = (dot(uemb_ref[:], W1_ref[0:EMB, :])
         + dot(g, W1_ref[EMB * 2:EMB * 2 + FEAT, :])
         + dot(a, W1_ref[EMB * 2 + FEAT:EMB * 2 + 2 * FEAT, :])
         + dot(o, W1_ref[EMB * 2 + 2 * FEAT:EMB * 2 + 3 * FEAT, :])
         + b1_ref[:][None, :])

    hm = hmean_ref[:].reshape(B2, 2 * EMB)
    pp = pos_ref[:].reshape(B2, 2 * EMB)
    nn = neg_ref[:].reshape(B2, 2 * EMB)
    W1h = W1_ref[EMB:EMB * 2, :]

    def tower(u1h, hmean):
        hidden = jnp.maximum(dot(u1h, T) + dot(hmean, W1h), 0.0)
        return dot(hidden, W2_ref[:]) + b2_ref[:][None, :]

    uv_e = tower(onehot(ue_ref[:], NUM_USERS), hm[:, 0:EMB])
    uv_o = tower(onehot(uo_ref[:], NUM_USERS), hm[:, EMB:2 * EMB])
    pe_out[:] = jnp.sum(uv_e * pp[:, 0:EMB], axis=1)
    po_out[:] = jnp.sum(uv_o * pp[:, EMB:2 * EMB], axis=1)
    ne_out[:] = jnp.sum(uv_e * nn[:, 0:EMB], axis=1)
    no_out[:] = jnp.sum(uv_o * nn[:, EMB:2 * EMB], axis=1)


def kernel(user_ids, history_item_ids, history_mask, pos_item_ids,
           neg_item_ids, user_emb, item_emb, gender_emb, age_emb, occ_emb,
           W1, b1, W2, b2, gender_ids, age_ids, occupation_ids):
    f32 = jnp.float32
    pad = H_STAGE - H
    pad_idx = (jnp.arange(B * pad, dtype=jnp.int32) % jnp.int32(99991)
               ).reshape(B, pad)
    idx_pad = jnp.concatenate([history_item_ids, pad_idx], axis=1)
    mask_pad = jnp.pad(history_mask, ((0, 0), (0, pad)))

    hist_sum, pos_rows, neg_rows = _sc_pool(
        idx_pad, mask_pad, pos_item_ids, neg_item_ids,
        item_emb)

    pe, po, ne, no = pl.pallas_call(
        _tc_body,
        out_shape=[jax.ShapeDtypeStruct((B // 2,), f32) for _ in range(4)],
    )(user_ids[0::2], user_ids[1::2], hist_sum.reshape(B * EMB),
      pos_rows.reshape(B * EMB), neg_rows.reshape(B * EMB),
      user_emb, gender_emb, age_emb, occ_emb,
      gender_ids, age_ids, occupation_ids, W1, b1, W2, b2)
    pos_s = jnp.stack([pe, po], axis=1).reshape(B)
    neg_s = jnp.stack([ne, no], axis=1).reshape(B)
    return pos_s, neg_s
